# baseline (speedup 1.0000x reference)
import jax
import jax.numpy as jnp
import numpy as np
from jax import lax
from jax.experimental import pallas as pl
from jax.experimental.pallas import tpu as pltpu
from jax.experimental.pallas import tpu_sc as plsc

N_NODES = 10000
N_EDGES = 160000
C = 128
AVG_NUM_NEIGHBORS = 16.0

def _act_norm_const():
    x = np.linspace(-12.0, 12.0, 48001)
    pdf = np.exp(-0.5 * x ** 2) / np.sqrt(2.0 * np.pi)
    phi = x / (1.0 + np.exp(-x))
    return float(np.sqrt(np.trapz(phi ** 2 * pdf, x)))

_INV_SILU_C = 1.0 / _act_norm_const()

_SLAB_A = 61440
_SLAB_B = N_EDGES - _SLAB_A
_EB = 1280


def _msg_body(ef_ref, eat_ref, w1_ref, w2_ref, w3_ref, out_ref):
    ef = ef_ref[...]
    h = jnp.dot(ef, w1_ref[...], preferred_element_type=jnp.float32)
    h = jax.nn.silu(h * (1.0 / np.sqrt(128.0))) * _INV_SILU_C
    h = jnp.dot(h, w2_ref[...], preferred_element_type=jnp.float32)
    h = jax.nn.silu(h * 0.125) * _INV_SILU_C
    mix = jnp.dot(h, w3_ref[...], preferred_element_type=jnp.float32) * 0.125
    scale = 1.0 / np.sqrt(AVG_NUM_NEIGHBORS)
    ea = jnp.transpose(eat_ref[...])
    base0 = ef * mix[:, :C] * scale
    base1 = ef * mix[:, C:] * scale
    out_ref[0] = base0 * ea[:, 0:1]
    out_ref[1] = base1 * ea[:, 1:2]
    out_ref[2] = base1 * ea[:, 2:3]
    out_ref[3] = base1 * ea[:, 3:4]


def _messages(edge_feats, edge_attrs_t, W1, W2, W3, n_rows, blk_off):
    grid = n_rows // _EB
    out = pl.pallas_call(
        _msg_body,
        grid=(grid,),
        in_specs=[
            pl.BlockSpec((_EB, C), lambda i: (blk_off + i, 0)),
            pl.BlockSpec((4, _EB), lambda i: (0, blk_off + i)),
            pl.BlockSpec((C, 64), lambda i: (0, 0)),
            pl.BlockSpec((64, 64), lambda i: (0, 0)),
            pl.BlockSpec((64, 4 * 64), lambda i: (0, 0)),
        ],
        out_specs=pl.BlockSpec((4, _EB, C), lambda i: (0, i, 0)),
        out_shape=jax.ShapeDtypeStruct((4, n_rows, C), jnp.float32),
    )(edge_feats, edge_attrs_t, W1, W2, W3)
    return out.reshape(4 * n_rows, C)


_NS = 16
_CHUNK = 128
_ROWS_PER_SUB = 624
_ROW_TAIL = N_NODES - _NS * _ROWS_PER_SUB
_ROW_TAIL0 = _NS * _ROWS_PER_SUB


def _make_scatter_body(e_per_sub, n_full, rem, seeded):
    assert n_full % 2 == 0 and n_full >= 4 and rem % 16 == 0

    def body(*refs):
        if seeded:
            (msg_hbm, recv_hbm, seed_hbm, out_hbm, acc, rsub,
             mb0, mb1, ib0, ib1, idxbuf_r,
             ls0, ls1, ss0, ss1) = refs
        else:
            (msg_hbm, recv_hbm, out_hbm, acc, rsub,
             mb0, mb1, ib0, ib1, idxbuf_r,
             ls0, ls1, ss0, ss1) = refs
        mbs = (mb0, mb1)
        ibs = (ib0, ib1)
        lss = (ls0, ls1)
        sss = (ss0, ss1)
        mbuf = mb0
        sem = ls0
        cid = lax.axis_index("c")
        sid = lax.axis_index("s")
        e_base = sid * e_per_sub

        pltpu.sync_copy(recv_hbm.at[pl.ds(e_base, e_per_sub)], rsub)

        if rem:
            for j in range(rem // 16):
                idxbuf_r[pl.ds(j * 16, 16)] = rsub[
                    pl.ds(n_full * _CHUNK + j * 16, 16)]

        row0 = sid * _ROWS_PER_SUB
        for p in range(2):
            grp = cid * 2 + p
            gm0 = grp * (e_per_sub * _NS)
            gn0 = grp * N_NODES
            if seeded:
                pltpu.sync_copy(
                    seed_hbm.at[pl.ds(gn0 + row0, _ROWS_PER_SUB)],
                    acc.at[pl.ds(row0, _ROWS_PER_SUB)])
                @pl.when(sid == 0)
                def _seed_tail():
                    pltpu.sync_copy(
                        seed_hbm.at[pl.ds(gn0 + _ROW_TAIL0, _ROW_TAIL)],
                        acc.at[pl.ds(_ROW_TAIL0, _ROW_TAIL)])
            else:
                def _zrow(i, _):
                    for j in range(C // 16):
                        mbuf[i, pl.ds(j * 16, 16)] = jnp.zeros(
                            (16,), jnp.float32)
                    return 0
                lax.fori_loop(0, _CHUNK, _zrow, 0)
                nz = _ROWS_PER_SUB // _CHUNK
                zrem = _ROWS_PER_SUB - nz * _CHUNK
                zcopies = [
                    (mbuf, acc.at[pl.ds(row0 + r * _CHUNK, _CHUNK)])
                    for r in range(nz)
                ] + ([(mbuf.at[pl.ds(0, zrem)],
                       acc.at[pl.ds(row0 + nz * _CHUNK, zrem)])]
                     if zrem else [])
                for src, dst in zcopies:
                    pltpu.async_copy(src, dst, sem)
                @pl.when(sid == 0)
                def _zero_tail():
                    pltpu.sync_copy(mbuf.at[pl.ds(0, _ROW_TAIL)],
                                    acc.at[pl.ds(_ROW_TAIL0, _ROW_TAIL)])
                for src, dst in zcopies:
                    pltpu.make_async_copy(src, dst, sem).wait()
            plsc.subcore_barrier()

            def _slab(ch):
                return msg_hbm.at[
                    pl.ds(gm0 + e_base + ch * _CHUNK, _CHUNK)]

            def _fill(ib, ch):
                for j in range(_CHUNK // 16):
                    ib[pl.ds(j * 16, 16)] = rsub[
                        pl.ds(ch * _CHUNK + j * 16, 16)]

            def _wait_scat(b):
                pltpu.make_async_copy(
                    mbs[b], acc.at[pl.ds(0, _CHUNK)], sss[b]).wait()

            def _slot(ch, b, first_pair):
                mb, ib = mbs[b], ibs[b]
                pltpu.make_async_copy(_slab(ch), mb, lss[b]).wait()
                _fill(ib, ch)
                pltpu.async_copy(mb, acc.at[ib], sss[b], add=True)
                b2 = 1 - b
                if b == 1:
                    _wait_scat(b2)
                else:
                    @pl.when(first_pair != 0)
                    def _drain_prev():
                        _wait_scat(b2)
                @pl.when(ch + 1 < n_full)
                def _load_ahead():
                    pltpu.async_copy(_slab(ch + 1), mbs[b2], lss[b2])

            pltpu.async_copy(_slab(0), mb0, ls0)

            def _pair(i, _):
                for b in range(2):
                    _slot(i * 2 + b, b, i)
                return 0
            lax.fori_loop(0, n_full // 2, _pair, 0)
            _wait_scat((n_full - 1) % 2)

            if rem:
                e0 = gm0 + e_base + n_full * _CHUNK
                pltpu.sync_copy(msg_hbm.at[pl.ds(e0, rem)],
                                mb0.at[pl.ds(0, rem)])
                pltpu.sync_copy(mb0.at[pl.ds(0, rem)],
                                acc.at[idxbuf_r], add=True)

            plsc.subcore_barrier()
            pltpu.sync_copy(acc.at[pl.ds(row0, _ROWS_PER_SUB)],
                            out_hbm.at[pl.ds(gn0 + row0, _ROWS_PER_SUB)])
            @pl.when(sid == 0)
            def _write_tail():
                pltpu.sync_copy(
                    acc.at[pl.ds(_ROW_TAIL0, _ROW_TAIL)],
                    out_hbm.at[pl.ds(gn0 + _ROW_TAIL0, _ROW_TAIL)])
            plsc.subcore_barrier()
    return body


def _scatter(messages, receivers, seed, e_per_sub, n_full, rem):
    mesh = plsc.VectorSubcoreMesh(core_axis_name="c", subcore_axis_name="s")
    rem_sz = max(rem, 16)
    f = pl.kernel(
        _make_scatter_body(e_per_sub, n_full, rem, seed is not None),
        out_type=jax.ShapeDtypeStruct((4 * N_NODES, C), jnp.float32),
        mesh=mesh,
        scratch_types=(
            [pltpu.VMEM_SHARED((N_NODES, C), jnp.float32)]
            + [pltpu.VMEM((e_per_sub,), jnp.int32)]
            + [pltpu.VMEM((_CHUNK, C), jnp.float32) for _ in range(2)]
            + [pltpu.VMEM((_CHUNK,), jnp.int32) for _ in range(2)]
            + [pltpu.VMEM((rem_sz,), jnp.int32)]
            + [pltpu.SemaphoreType.DMA for _ in range(4)]
        ),
    )
    if seed is not None:
        return f(messages, receivers, seed)
    return f(messages, receivers)


def _perm_matrix():
    p = np.zeros((3 * C, 3 * C), dtype=np.float32)
    for k in range(3):
        for c in range(C):
            p[k * C + c, 3 * c + k] = 1.0
    return p

_PERM = _perm_matrix()
_NB = 1000


def _final_body(g0_ref, g1_ref, g2_ref, g3_ref, p_ref, out_ref):
    out_ref[:, 0:C] = g0_ref[0]
    vec = jnp.concatenate([g1_ref[0], g2_ref[0], g3_ref[0]], axis=1)
    out_ref[:, C:] = jnp.dot(vec, p_ref[...],
                             preferred_element_type=jnp.float32,
                             precision=lax.Precision.HIGHEST)


def _finalize(acc):
    acc4 = acc.reshape(4, N_NODES, C)
    gspec = [pl.BlockSpec((1, _NB, C), lambda i, g=g: (g, i, 0))
             for g in range(4)]
    return pl.pallas_call(
        _final_body,
        grid=(N_NODES // _NB,),
        in_specs=gspec + [pl.BlockSpec((3 * C, 3 * C), lambda i: (0, 0))],
        out_specs=pl.BlockSpec((_NB, 4 * C), lambda i: (i, 0)),
        out_shape=jax.ShapeDtypeStruct((N_NODES, 4 * C), jnp.float32),
    )(acc4, acc4, acc4, acc4, jnp.asarray(_PERM))


def kernel(edge_feats, edge_attrs, receivers, n_nodes, W1, W2, W3):
    recv = receivers.astype(jnp.int32)
    ea_t = jnp.transpose(edge_attrs)
    msg_a = _messages(edge_feats, ea_t, W1, W2, W3, _SLAB_A, 0)
    msg_b = _messages(edge_feats, ea_t, W1, W2, W3, _SLAB_B,
                      _SLAB_A // _EB)
    part = _scatter(msg_a, lax.slice(recv, (0,), (_SLAB_A,)), None,
                    _SLAB_A // _NS, _SLAB_A // _NS // _CHUNK, 0)
    acc = _scatter(msg_b, lax.slice(recv, (_SLAB_A,), (N_EDGES,)), part,
                   _SLAB_B // _NS, _SLAB_B // _NS // _CHUNK,
                   _SLAB_B // _NS - (_SLAB_B // _NS // _CHUNK) * _CHUNK)
    return _finalize(acc)


# speedup vs baseline: 1.0706x; 1.0706x over previous
import jax
import jax.numpy as jnp
import numpy as np
from jax import lax
from jax.experimental import pallas as pl
from jax.experimental.pallas import tpu as pltpu
from jax.experimental.pallas import tpu_sc as plsc

N_NODES = 10000
N_EDGES = 160000
C = 128
AVG_NUM_NEIGHBORS = 16.0

def _act_norm_const():
    x = np.linspace(-12.0, 12.0, 48001)
    pdf = np.exp(-0.5 * x ** 2) / np.sqrt(2.0 * np.pi)
    phi = x / (1.0 + np.exp(-x))
    return float(np.sqrt(np.trapz(phi ** 2 * pdf, x)))

_INV_SILU_C = 1.0 / _act_norm_const()

_SLAB_A = 61440
_SLAB_B = N_EDGES - _SLAB_A
_EB = 1280


def _msg_body(ef_ref, eat_ref, w1_ref, w2_ref, w3_ref, out_ref):
    ef = ef_ref[...]
    h = jnp.dot(ef, w1_ref[...], preferred_element_type=jnp.float32)
    h = jax.nn.silu(h * (1.0 / np.sqrt(128.0))) * _INV_SILU_C
    h = jnp.dot(h, w2_ref[...], preferred_element_type=jnp.float32)
    h = jax.nn.silu(h * 0.125) * _INV_SILU_C
    mix = jnp.dot(h, w3_ref[...], preferred_element_type=jnp.float32) * 0.125
    scale = 1.0 / np.sqrt(AVG_NUM_NEIGHBORS)
    ea = jnp.transpose(eat_ref[...])
    base0 = ef * mix[:, :C] * scale
    base1 = ef * mix[:, C:] * scale
    out_ref[0] = base0 * ea[:, 0:1]
    out_ref[1] = base1 * ea[:, 1:2]
    out_ref[2] = base1 * ea[:, 2:3]
    out_ref[3] = base1 * ea[:, 3:4]


def _messages(edge_feats, edge_attrs_t, W1, W2, W3, n_rows, blk_off):
    grid = n_rows // _EB
    out = pl.pallas_call(
        _msg_body,
        grid=(grid,),
        in_specs=[
            pl.BlockSpec((_EB, C), lambda i: (blk_off + i, 0)),
            pl.BlockSpec((4, _EB), lambda i: (0, blk_off + i)),
            pl.BlockSpec((C, 64), lambda i: (0, 0)),
            pl.BlockSpec((64, 64), lambda i: (0, 0)),
            pl.BlockSpec((64, 4 * 64), lambda i: (0, 0)),
        ],
        out_specs=pl.BlockSpec((4, _EB, C), lambda i: (0, i, 0)),
        out_shape=jax.ShapeDtypeStruct((4, n_rows, C), jnp.float32),
    )(edge_feats, edge_attrs_t, W1, W2, W3)
    return out.reshape(4 * n_rows, C)


_NS = 16
_CHUNK = 128
_ROWS_PER_SUB = 624
_ROW_TAIL = N_NODES - _NS * _ROWS_PER_SUB
_ROW_TAIL0 = _NS * _ROWS_PER_SUB


def _make_scatter_body(e_per_sub, n_full, rem, seeded):
    assert n_full % 2 == 0 and n_full >= 4 and rem % 16 == 0

    def body(*refs):
        if seeded:
            (msg_hbm, recv_hbm, seed_hbm, out_hbm, acc, rsub,
             mb0, mb1, ib0, ib1, idxbuf_r,
             ls0, ls1, ss0, ss1) = refs
        else:
            (msg_hbm, recv_hbm, out_hbm, acc, rsub,
             mb0, mb1, ib0, ib1, idxbuf_r,
             ls0, ls1, ss0, ss1) = refs
        mbs = (mb0, mb1)
        ibs = (ib0, ib1)
        lss = (ls0, ls1)
        sss = (ss0, ss1)
        mbuf = mb0
        sem = ls0
        cid = lax.axis_index("c")
        sid = lax.axis_index("s")
        e_base = sid * e_per_sub

        pltpu.sync_copy(recv_hbm.at[pl.ds(e_base, e_per_sub)], rsub)

        if rem:
            for j in range(rem // 16):
                idxbuf_r[pl.ds(j * 16, 16)] = rsub[
                    pl.ds(n_full * _CHUNK + j * 16, 16)]

        row0 = sid * _ROWS_PER_SUB
        for p in range(2):
            grp = cid * 2 + p
            gm0 = grp * (e_per_sub * _NS)
            gn0 = grp * N_NODES
            if seeded:
                pltpu.sync_copy(
                    seed_hbm.at[pl.ds(gn0 + row0, _ROWS_PER_SUB)],
                    acc.at[pl.ds(row0, _ROWS_PER_SUB)])
                @pl.when(sid == 0)
                def _seed_tail():
                    pltpu.sync_copy(
                        seed_hbm.at[pl.ds(gn0 + _ROW_TAIL0, _ROW_TAIL)],
                        acc.at[pl.ds(_ROW_TAIL0, _ROW_TAIL)])
            else:
                def _zrow(i, _):
                    for j in range(C // 16):
                        mbuf[i, pl.ds(j * 16, 16)] = jnp.zeros(
                            (16,), jnp.float32)
                    return 0
                lax.fori_loop(0, _CHUNK, _zrow, 0)
                nz = _ROWS_PER_SUB // _CHUNK
                zrem = _ROWS_PER_SUB - nz * _CHUNK
                zcopies = [
                    (mbuf, acc.at[pl.ds(row0 + r * _CHUNK, _CHUNK)])
                    for r in range(nz)
                ] + ([(mbuf.at[pl.ds(0, zrem)],
                       acc.at[pl.ds(row0 + nz * _CHUNK, zrem)])]
                     if zrem else [])
                for src, dst in zcopies:
                    pltpu.async_copy(src, dst, sem)
                @pl.when(sid == 0)
                def _zero_tail():
                    pltpu.sync_copy(mbuf.at[pl.ds(0, _ROW_TAIL)],
                                    acc.at[pl.ds(_ROW_TAIL0, _ROW_TAIL)])
                for src, dst in zcopies:
                    pltpu.make_async_copy(src, dst, sem).wait()
            plsc.subcore_barrier()

            def _slab(ch):
                return msg_hbm.at[
                    pl.ds(gm0 + e_base + ch * _CHUNK, _CHUNK)]

            def _fill(ib, ch):
                for j in range(_CHUNK // 16):
                    ib[pl.ds(j * 16, 16)] = rsub[
                        pl.ds(ch * _CHUNK + j * 16, 16)]

            pltpu.async_copy(_slab(0), mb0, ls0)
            pltpu.async_copy(_slab(1), mb1, ls1)

            def _pair(i, _):
                for b in range(2):
                    mb, ib = mbs[b], ibs[b]
                    ch = i * 2 + b
                    pltpu.make_async_copy(_slab(ch), mb, lss[b]).wait()
                    _fill(ib, ch)
                    pltpu.sync_copy(mb, acc.at[ib], add=True)
                    @pl.when(ch + 2 < n_full)
                    def _load_ahead():
                        pltpu.async_copy(_slab(ch + 2), mb, lss[b])
                return 0
            lax.fori_loop(0, n_full // 2, _pair, 0)

            if rem:
                e0 = gm0 + e_base + n_full * _CHUNK
                pltpu.sync_copy(msg_hbm.at[pl.ds(e0, rem)],
                                mb0.at[pl.ds(0, rem)])
                pltpu.sync_copy(mb0.at[pl.ds(0, rem)],
                                acc.at[idxbuf_r], add=True)

            plsc.subcore_barrier()
            pltpu.sync_copy(acc.at[pl.ds(row0, _ROWS_PER_SUB)],
                            out_hbm.at[pl.ds(gn0 + row0, _ROWS_PER_SUB)])
            @pl.when(sid == 0)
            def _write_tail():
                pltpu.sync_copy(
                    acc.at[pl.ds(_ROW_TAIL0, _ROW_TAIL)],
                    out_hbm.at[pl.ds(gn0 + _ROW_TAIL0, _ROW_TAIL)])
            plsc.subcore_barrier()
    return body


def _scatter(messages, receivers, seed, e_per_sub, n_full, rem):
    mesh = plsc.VectorSubcoreMesh(core_axis_name="c", subcore_axis_name="s")
    rem_sz = max(rem, 16)
    f = pl.kernel(
        _make_scatter_body(e_per_sub, n_full, rem, seed is not None),
        out_type=jax.ShapeDtypeStruct((4 * N_NODES, C), jnp.float32),
        mesh=mesh,
        scratch_types=(
            [pltpu.VMEM_SHARED((N_NODES, C), jnp.float32)]
            + [pltpu.VMEM((e_per_sub,), jnp.int32)]
            + [pltpu.VMEM((_CHUNK, C), jnp.float32) for _ in range(2)]
            + [pltpu.VMEM((_CHUNK,), jnp.int32) for _ in range(2)]
            + [pltpu.VMEM((rem_sz,), jnp.int32)]
            + [pltpu.SemaphoreType.DMA for _ in range(4)]
        ),
    )
    if seed is not None:
        return f(messages, receivers, seed)
    return f(messages, receivers)


def _perm_matrix():
    p = np.zeros((3 * C, 3 * C), dtype=np.float32)
    for k in range(3):
        for c in range(C):
            p[k * C + c, 3 * c + k] = 1.0
    return p

_PERM = _perm_matrix()
_NB = 1000


def _final_body(g0_ref, g1_ref, g2_ref, g3_ref, p_ref, out_ref):
    out_ref[:, 0:C] = g0_ref[0]
    vec = jnp.concatenate([g1_ref[0], g2_ref[0], g3_ref[0]], axis=1)
    out_ref[:, C:] = jnp.dot(vec, p_ref[...],
                             preferred_element_type=jnp.float32,
                             precision=lax.Precision.HIGHEST)


def _finalize(acc):
    acc4 = acc.reshape(4, N_NODES, C)
    gspec = [pl.BlockSpec((1, _NB, C), lambda i, g=g: (g, i, 0))
             for g in range(4)]
    return pl.pallas_call(
        _final_body,
        grid=(N_NODES // _NB,),
        in_specs=gspec + [pl.BlockSpec((3 * C, 3 * C), lambda i: (0, 0))],
        out_specs=pl.BlockSpec((_NB, 4 * C), lambda i: (i, 0)),
        out_shape=jax.ShapeDtypeStruct((N_NODES, 4 * C), jnp.float32),
    )(acc4, acc4, acc4, acc4, jnp.asarray(_PERM))


def kernel(edge_feats, edge_attrs, receivers, n_nodes, W1, W2, W3):
    recv = receivers.astype(jnp.int32)
    ea_t = jnp.transpose(edge_attrs)
    msg_a = _messages(edge_feats, ea_t, W1, W2, W3, _SLAB_A, 0)
    msg_b = _messages(edge_feats, ea_t, W1, W2, W3, _SLAB_B,
                      _SLAB_A // _EB)
    part = _scatter(msg_a, lax.slice(recv, (0,), (_SLAB_A,)), None,
                    _SLAB_A // _NS, _SLAB_A // _NS // _CHUNK, 0)
    acc = _scatter(msg_b, lax.slice(recv, (_SLAB_A,), (N_EDGES,)), part,
                   _SLAB_B // _NS, _SLAB_B // _NS // _CHUNK,
                   _SLAB_B // _NS - (_SLAB_B // _NS // _CHUNK) * _CHUNK)
    return _finalize(acc)
